# baseline (speedup 1.0000x reference)
import functools

import jax
import jax.numpy as jnp
from jax import lax
from jax.experimental import pallas as pl
from jax.experimental.pallas import tpu as pltpu
from jax.experimental.pallas import tpu_sc as plsc

N_FIELDS = 26
VOCAB = 100
DIM = 128
BATCH = 16384

NC = 2
NS = 16
NW = NC * NS
L = 16

ELEMS_PER_W = BATCH // NW
NE = 4
ITERS = ELEMS_PER_W // NE
CROWS = NE * N_FIELDS


def _sc_embed(xflat, U):
  mesh = plsc.VectorSubcoreMesh(core_axis_name="c", subcore_axis_name="s")

  @functools.partial(
      pl.kernel,
      mesh=mesh,
      out_type=jax.ShapeDtypeStruct((BATCH, DIM, N_FIELDS), jnp.float32),
      scratch_types=[
          pltpu.VMEM((CROWS,), jnp.int32),
          pltpu.VMEM((CROWS,), jnp.int32),
          pltpu.VMEM((CROWS, DIM), jnp.float32),
          pltpu.VMEM((CROWS, DIM), jnp.float32),
          pltpu.VMEM((NE, DIM, N_FIELDS), jnp.float32),
          pltpu.SemaphoreType.DMA,
          pltpu.SemaphoreType.DMA,
          pltpu.SemaphoreType.DMA,
      ],
      compiler_params=pltpu.CompilerParams(
          needs_layout_passes=False),
  )
  def k(x_hbm, u_hbm, o_hbm, idxA, idxB, rowsA, rowsB, outA,
        g0, g1, o0):
    idx_ = (idxA, idxB)
    rows_ = (rowsA, rowsB)
    gsem = (g0, g1)
    wid = lax.axis_index("s") * NC + lax.axis_index("c")
    ebase0 = wid * ELEMS_PER_W
    lane = lax.iota(jnp.int32, L)
    zero = lane * 0

    def prefetch(t, b):
      ebase = ebase0 + t * NE
      pltpu.sync_copy(x_hbm.at[pl.ds(ebase * N_FIELDS, CROWS)], idx_[b])
      for j in range(CROWS // L):
        fld = lax.rem(j * L + lane, N_FIELDS) * VOCAB
        sl = pl.ds(j * L, L)
        idx_[b][sl] = idx_[b][sl] + fld
      pltpu.make_async_copy(u_hbm.at[idx_[b]], rows_[b], gsem[b]).start()

    def wait_gather(b):
      pltpu.make_async_copy(u_hbm.at[idx_[b]], rows_[b], gsem[b]).wait()

    def out_copy(t):
      ebase = ebase0 + t * NE
      return pltpu.make_async_copy(
          outA, o_hbm.at[pl.ds(ebase, NE)], o0)

    def transpose(b):
      rows_b = rows_[b]

      @plsc.parallel_loop(0, NE)
      def _(e):
        rowb = e * N_FIELDS
        e_idx = zero + e
        for c in range(N_FIELDS):
          row = rowb + c
          c_idx = zero + c
          for db in range(DIM // L):
            v = rows_b[row, pl.ds(db * L, L)]
            plsc.store_scatter(outA, [e_idx, db * L + lane, c_idx], v)

    prefetch(0, 0)
    prefetch(1, 1)

    def body(tt, carry):
      for b in range(2):
        t = tt * 2 + b
        wait_gather(b)

        @pl.when(t > 0)
        def _():
          out_copy(t - 1).wait()

        transpose(b)
        out_copy(t).start()

        @pl.when(t + 2 < ITERS)
        def _():
          prefetch(t + 2, b)

      return carry

    lax.fori_loop(0, ITERS // 2, body, 0)
    out_copy(ITERS - 1).wait()

  return k(xflat, U)


def kernel(x, W):
  xflat = x.reshape(-1).astype(jnp.int32)
  U = W.reshape(N_FIELDS * VOCAB, DIM)
  return _sc_embed(xflat, U)


# speedup vs baseline: 9.0729x; 9.0729x over previous
import functools

import jax
import jax.numpy as jnp
from jax import lax
from jax.experimental import pallas as pl
from jax.experimental.pallas import tpu as pltpu
from jax.experimental.pallas import tpu_sc as plsc

N_FIELDS = 26
VOCAB = 100
DIM = 128
BATCH = 16384
LOG2_BATCH = 14

NC = 2
NS = 16
NW = NC * NS
L = 16

ROWS = N_FIELDS * BATCH
ROWS_PER_W = ROWS // NW
CHUNK = 128
ITERS = ROWS_PER_W // CHUNK


def _sc_embed(xt, U):
  mesh = plsc.VectorSubcoreMesh(core_axis_name="c", subcore_axis_name="s")

  @functools.partial(
      pl.kernel,
      mesh=mesh,
      out_type=jax.ShapeDtypeStruct((ROWS, DIM), jnp.float32),
      scratch_types=[
          pltpu.VMEM((CHUNK,), jnp.int32),
          pltpu.VMEM((CHUNK,), jnp.int32),
          pltpu.VMEM((CHUNK, DIM), jnp.float32),
          pltpu.VMEM((CHUNK, DIM), jnp.float32),
          pltpu.SemaphoreType.DMA,
          pltpu.SemaphoreType.DMA,
          pltpu.SemaphoreType.DMA,
          pltpu.SemaphoreType.DMA,
      ],
  )
  def k(x_hbm, u_hbm, o_hbm, idxA, idxB, rowsA, rowsB, g0, g1, o0, o1):
    idx_ = (idxA, idxB)
    rows_ = (rowsA, rowsB)
    gsem = (g0, g1)
    osem = (o0, o1)
    wid = lax.axis_index("s") * NC + lax.axis_index("c")
    rbase0 = wid * ROWS_PER_W
    lane = lax.iota(jnp.int32, L)

    def gather(t, b):
      off = rbase0 + t * CHUNK
      pltpu.sync_copy(x_hbm.at[pl.ds(off, CHUNK)], idx_[b])
      for j in range(CHUNK // L):
        pos = off + j * L + lane
        fld = lax.shift_right_logical(pos, LOG2_BATCH) * VOCAB
        sl = pl.ds(j * L, L)
        idx_[b][sl] = idx_[b][sl] + fld
      pltpu.make_async_copy(u_hbm.at[idx_[b]], rows_[b], gsem[b]).start()

    def wait_gather(b):
      pltpu.make_async_copy(u_hbm.at[idx_[b]], rows_[b], gsem[b]).wait()

    def out_copy(t, b):
      off = rbase0 + t * CHUNK
      return pltpu.make_async_copy(
          rows_[b], o_hbm.at[pl.ds(off, CHUNK)], osem[b])

    gather(0, 0)
    gather(1, 1)

    def body(tt, carry):
      for b in range(2):
        t = tt * 2 + b
        wait_gather(b)
        out_copy(t, b).start()

        @pl.when(t + 2 < ITERS)
        def _():
          out_copy(t, b).wait()
          gather(t + 2, b)

      return carry

    lax.fori_loop(0, ITERS // 2, body, 0)
    out_copy(ITERS - 2, 0).wait()
    out_copy(ITERS - 1, 1).wait()

  return k(xt, U)


def kernel(x, W):
  xt = jnp.transpose(x.astype(jnp.int32)).reshape(-1)
  U = W.reshape(N_FIELDS * VOCAB, DIM)
  E2 = _sc_embed(xt, U).reshape(N_FIELDS, BATCH, DIM)
  return jnp.transpose(E2, (1, 2, 0))


# speedup vs baseline: 12.9211x; 1.4241x over previous
import functools

import jax
import jax.numpy as jnp
from jax import lax
from jax.experimental import pallas as pl
from jax.experimental.pallas import tpu as pltpu
from jax.experimental.pallas import tpu_sc as plsc

N_FIELDS = 26
VOCAB = 100
DIM = 128
BATCH = 16384
LOG2_BATCH = 14

NC = 2
NS = 16
NW = NC * NS
L = 16

ROWS = N_FIELDS * BATCH
ROWS_PER_W = ROWS // NW
CHUNK = 128
ITERS = ROWS_PER_W // CHUNK


def _sc_embed(xt, U):
  mesh = plsc.VectorSubcoreMesh(core_axis_name="c", subcore_axis_name="s")

  @functools.partial(
      pl.kernel,
      mesh=mesh,
      out_type=jax.ShapeDtypeStruct((ROWS, DIM), jnp.float32),
      scratch_types=[
          pltpu.VMEM((CHUNK,), jnp.int32),
          pltpu.VMEM((CHUNK,), jnp.int32),
          pltpu.VMEM((CHUNK, DIM), jnp.float32),
          pltpu.VMEM((CHUNK, DIM), jnp.float32),
          pltpu.VMEM_SHARED((N_FIELDS * VOCAB, DIM), jnp.float32),
          pltpu.SemaphoreType.DMA,
          pltpu.SemaphoreType.DMA,
          pltpu.SemaphoreType.DMA,
          pltpu.SemaphoreType.DMA,
      ],
  )
  def k(x_hbm, u_hbm, o_hbm, idxA, idxB, rowsA, rowsB, u_sp,
        g0, g1, o0, o1):
    idx_ = (idxA, idxB)
    rows_ = (rowsA, rowsB)
    gsem = (g0, g1)
    osem = (o0, o1)
    wid = lax.axis_index("s") * NC + lax.axis_index("c")
    rbase0 = wid * ROWS_PER_W
    lane = lax.iota(jnp.int32, L)

    def gather(t, b):
      off = rbase0 + t * CHUNK
      pltpu.sync_copy(x_hbm.at[pl.ds(off, CHUNK)], idx_[b])
      for j in range(CHUNK // L):
        pos = off + j * L + lane
        fld = lax.shift_right_logical(pos, LOG2_BATCH) * VOCAB
        sl = pl.ds(j * L, L)
        idx_[b][sl] = idx_[b][sl] + fld
      pltpu.make_async_copy(u_sp.at[idx_[b]], rows_[b], gsem[b]).start()

    def wait_gather(b):
      pltpu.make_async_copy(u_sp.at[idx_[b]], rows_[b], gsem[b]).wait()

    def out_copy(t, b):
      off = rbase0 + t * CHUNK
      return pltpu.make_async_copy(
          rows_[b], o_hbm.at[pl.ds(off, CHUNK)], osem[b])

    @pl.when(lax.axis_index("s") == 0)
    def _():
      pltpu.sync_copy(u_hbm, u_sp)

    plsc.subcore_barrier()

    gather(0, 0)
    gather(1, 1)

    def body(tt, carry):
      for b in range(2):
        t = tt * 2 + b
        wait_gather(b)
        out_copy(t, b).start()

        @pl.when(t + 2 < ITERS)
        def _():
          out_copy(t, b).wait()
          gather(t + 2, b)

      return carry

    lax.fori_loop(0, ITERS // 2, body, 0)
    out_copy(ITERS - 2, 0).wait()
    out_copy(ITERS - 1, 1).wait()

  return k(xt, U)


def kernel(x, W):
  xt = jnp.transpose(x.astype(jnp.int32)).reshape(-1)
  U = W.reshape(N_FIELDS * VOCAB, DIM)
  E2 = _sc_embed(xt, U).reshape(N_FIELDS, BATCH, DIM)
  return jnp.transpose(E2, (1, 2, 0))


# speedup vs baseline: 18.5884x; 1.4386x over previous
import functools

import jax
import jax.numpy as jnp
from jax import lax
from jax.experimental import pallas as pl
from jax.experimental.pallas import tpu as pltpu
from jax.experimental.pallas import tpu_sc as plsc

N_FIELDS = 26
VOCAB = 100
DIM = 128
BATCH = 16384
LOG2_BATCH = 14

NC = 2
NS = 16
NW = NC * NS
L = 16

ROWS = N_FIELDS * BATCH
ROWS_PER_W = ROWS // NW
CHUNK = 128
ITERS = ROWS_PER_W // CHUNK
NB = 4


def _sc_embed(xt, U):
  mesh = plsc.VectorSubcoreMesh(core_axis_name="c", subcore_axis_name="s")

  @functools.partial(
      pl.kernel,
      mesh=mesh,
      out_type=jax.ShapeDtypeStruct((ROWS, DIM), jnp.float32),
      scratch_types=(
          [pltpu.VMEM((ROWS_PER_W,), jnp.int32)]
          + [pltpu.VMEM((CHUNK, DIM), jnp.float32)] * NB
          + [pltpu.VMEM_SHARED((N_FIELDS * VOCAB, DIM), jnp.float32)]
          + [pltpu.SemaphoreType.DMA] * (2 * NB + 1)
      ),
  )
  def k(x_hbm, u_hbm, o_hbm, xt_v, r0, r1, r2, r3, u_sp,
        g0, g1, g2, g3, o0, o1, o2, o3, xsem):
    rows_ = (r0, r1, r2, r3)
    gsem = (g0, g1, g2, g3)
    osem = (o0, o1, o2, o3)
    wid = lax.axis_index("s") * NC + lax.axis_index("c")
    rbase0 = wid * ROWS_PER_W
    lane = lax.iota(jnp.int32, L)

    @pl.when(lax.axis_index("s") == 0)
    def _():
      pltpu.sync_copy(u_hbm, u_sp)

    pltpu.make_async_copy(
        x_hbm.at[pl.ds(rbase0, ROWS_PER_W)], xt_v, xsem).start()
    pltpu.make_async_copy(
        x_hbm.at[pl.ds(rbase0, ROWS_PER_W)], xt_v, xsem).wait()

    @plsc.parallel_loop(0, ROWS_PER_W // L)
    def _(j):
      pos = rbase0 + j * L + lane
      fld = lax.shift_right_logical(pos, LOG2_BATCH) * VOCAB
      sl = pl.ds(j * L, L)
      xt_v[sl] = xt_v[sl] + fld

    plsc.subcore_barrier()

    def gather(t, b):
      pltpu.make_async_copy(
          u_sp.at[xt_v.at[pl.ds(t * CHUNK, CHUNK)]], rows_[b],
          gsem[b]).start()

    def wait_gather(t, b):
      pltpu.make_async_copy(
          u_sp.at[xt_v.at[pl.ds(t * CHUNK, CHUNK)]], rows_[b],
          gsem[b]).wait()

    def out_copy(t, b):
      off = rbase0 + t * CHUNK
      return pltpu.make_async_copy(
          rows_[b], o_hbm.at[pl.ds(off, CHUNK)], osem[b])

    for b in range(NB):
      gather(b, b)

    def body(tt, carry):
      for b in range(NB):
        t = tt * NB + b
        wait_gather(t, b)
        out_copy(t, b).start()

        @pl.when(t + NB < ITERS)
        def _():
          out_copy(t, b).wait()
          gather(t + NB, b)

      return carry

    lax.fori_loop(0, ITERS // NB, body, 0)
    for b in range(NB):
      out_copy(ITERS - NB + b, b).wait()

  return k(xt, U)


def kernel(x, W):
  xt = jnp.transpose(x.astype(jnp.int32)).reshape(-1)
  U = W.reshape(N_FIELDS * VOCAB, DIM)
  E2 = _sc_embed(xt, U).reshape(N_FIELDS, BATCH, DIM)
  return jnp.transpose(E2, (1, 2, 0))
